# Initial kernel scaffold; baseline (speedup 1.0000x reference)
import jax
import jax.numpy as jnp
from jax.experimental import pallas as pl


def kernel(hidden_states, gate_weight, e_score_correction_bias, w_gate, w_up, w_down, shared_gate, shared_up, shared_down):
    raise NotImplementedError("write your pallas kernel here")


# speedup vs baseline: 2.1156x; 2.1156x over previous
import jax
import jax.numpy as jnp
from jax.experimental import pallas as pl
from jax.experimental.pallas import tpu as pltpu

T = 2048
HIDDEN = 1024
N_EXPERTS = 8
TOP_K = 2
D_FF = 768
D_FF_SHARED = 1536


def _dot_t(a, b):
    return jax.lax.dot_general(a, b, (((1,), (1,)), ((), ())),
                               preferred_element_type=jnp.float32)


def _routed_kernel(x_ref, gw_ref, bias_ref, wg_ref, wu_ref, wd_ref,
                   out_ref, comb_ref):
    e = pl.program_id(0)
    x = x_ref[...]

    @pl.when(e == 0)
    def _router():
        logits = _dot_t(x, gw_ref[...])
        scores = jax.nn.sigmoid(logits)
        biased = scores + bias_ref[...]
        eiota = jax.lax.broadcasted_iota(jnp.int32, (T, N_EXPERTS), 1)
        m1 = jnp.max(biased, axis=1, keepdims=True)
        idx1 = jnp.min(jnp.where(biased == m1, eiota, N_EXPERTS),
                       axis=1, keepdims=True)
        oh1 = eiota == idx1
        b2 = jnp.where(oh1, -jnp.inf, biased)
        m2 = jnp.max(b2, axis=1, keepdims=True)
        idx2 = jnp.min(jnp.where(b2 == m2, eiota, N_EXPERTS),
                       axis=1, keepdims=True)
        oh2 = eiota == idx2
        w1 = jnp.sum(jnp.where(oh1, scores, 0.0), axis=1, keepdims=True)
        w2 = jnp.sum(jnp.where(oh2, scores, 0.0), axis=1, keepdims=True)
        den = w1 + w2 + 1e-20
        comb_ref[...] = (jnp.where(oh1, w1, 0.0)
                         + jnp.where(oh2, w2, 0.0)) / den

    combine = comb_ref[...]
    eids = jax.lax.broadcasted_iota(jnp.int32, (T, N_EXPERTS), 1)
    w_col = jnp.sum(jnp.where(eids == e, combine, 0.0),
                    axis=1, keepdims=True)

    g = _dot_t(x, wg_ref[0])
    u = _dot_t(x, wu_ref[0])
    h = (g * jax.nn.sigmoid(g)) * u
    y = _dot_t(h, wd_ref[0])
    contrib = y * w_col

    @pl.when(e == 0)
    def _init():
        out_ref[...] = contrib

    @pl.when(e > 0)
    def _acc():
        out_ref[...] += contrib


def _shared_kernel(x_ref, sg_ref, su_ref, sd_ref, r_ref, o_ref):
    x = x_ref[...]
    g = _dot_t(x, sg_ref[...])
    u = _dot_t(x, su_ref[...])
    h = (g * jax.nn.sigmoid(g)) * u
    y = _dot_t(h, sd_ref[...])
    o_ref[...] = r_ref[...] + y


def kernel(hidden_states, gate_weight, e_score_correction_bias,
           w_gate, w_up, w_down, shared_gate, shared_up, shared_down):
    x = hidden_states
    bias2d = e_score_correction_bias.reshape(1, N_EXPERTS)

    routed = pl.pallas_call(
        _routed_kernel,
        grid=(N_EXPERTS,),
        in_specs=[
            pl.BlockSpec((T, HIDDEN), lambda e: (0, 0)),
            pl.BlockSpec((N_EXPERTS, HIDDEN), lambda e: (0, 0)),
            pl.BlockSpec((1, N_EXPERTS), lambda e: (0, 0)),
            pl.BlockSpec((1, D_FF, HIDDEN), lambda e: (e, 0, 0)),
            pl.BlockSpec((1, D_FF, HIDDEN), lambda e: (e, 0, 0)),
            pl.BlockSpec((1, HIDDEN, D_FF), lambda e: (e, 0, 0)),
        ],
        out_specs=pl.BlockSpec((T, HIDDEN), lambda e: (0, 0)),
        out_shape=jax.ShapeDtypeStruct((T, HIDDEN), jnp.float32),
        scratch_shapes=[pltpu.VMEM((T, N_EXPERTS), jnp.float32)],
        compiler_params=pltpu.CompilerParams(
            dimension_semantics=("arbitrary",)),
    )(x, gate_weight, bias2d, w_gate, w_up, w_down)

    TB = 512
    out = pl.pallas_call(
        _shared_kernel,
        grid=(T // TB,),
        in_specs=[
            pl.BlockSpec((TB, HIDDEN), lambda i: (i, 0)),
            pl.BlockSpec((D_FF_SHARED, HIDDEN), lambda i: (0, 0)),
            pl.BlockSpec((D_FF_SHARED, HIDDEN), lambda i: (0, 0)),
            pl.BlockSpec((HIDDEN, D_FF_SHARED), lambda i: (0, 0)),
            pl.BlockSpec((TB, HIDDEN), lambda i: (i, 0)),
        ],
        out_specs=pl.BlockSpec((TB, HIDDEN), lambda i: (i, 0)),
        out_shape=jax.ShapeDtypeStruct((T, HIDDEN), jnp.float32),
        compiler_params=pltpu.CompilerParams(
            dimension_semantics=("parallel",)),
    )(x, shared_gate, shared_up, shared_down, routed)

    return out


# speedup vs baseline: 2.1415x; 1.0123x over previous
import jax
import jax.numpy as jnp
from jax.experimental import pallas as pl
from jax.experimental.pallas import tpu as pltpu

T = 2048
HIDDEN = 1024
N_EXPERTS = 8
TOP_K = 2
D_FF = 768
D_FF_SHARED = 1536


def _dot_t(a, b):
    return jax.lax.dot_general(a.astype(jnp.bfloat16), b.astype(jnp.bfloat16),
                               (((1,), (1,)), ((), ())),
                               preferred_element_type=jnp.float32)


def _routed_kernel(x_ref, gw_ref, bias_ref, wg_ref, wu_ref, wd_ref,
                   out_ref, comb_ref):
    e = pl.program_id(0)
    x = x_ref[...]

    @pl.when(e == 0)
    def _router():
        logits = _dot_t(x, gw_ref[...])
        scores = jax.nn.sigmoid(logits)
        biased = scores + bias_ref[...]
        eiota = jax.lax.broadcasted_iota(jnp.int32, (T, N_EXPERTS), 1)
        m1 = jnp.max(biased, axis=1, keepdims=True)
        idx1 = jnp.min(jnp.where(biased == m1, eiota, N_EXPERTS),
                       axis=1, keepdims=True)
        oh1 = eiota == idx1
        b2 = jnp.where(oh1, -jnp.inf, biased)
        m2 = jnp.max(b2, axis=1, keepdims=True)
        idx2 = jnp.min(jnp.where(b2 == m2, eiota, N_EXPERTS),
                       axis=1, keepdims=True)
        oh2 = eiota == idx2
        w1 = jnp.sum(jnp.where(oh1, scores, 0.0), axis=1, keepdims=True)
        w2 = jnp.sum(jnp.where(oh2, scores, 0.0), axis=1, keepdims=True)
        den = w1 + w2 + 1e-20
        comb_ref[...] = (jnp.where(oh1, w1, 0.0)
                         + jnp.where(oh2, w2, 0.0)) / den

    combine = comb_ref[...]
    eids = jax.lax.broadcasted_iota(jnp.int32, (T, N_EXPERTS), 1)
    w_col = jnp.sum(jnp.where(eids == e, combine, 0.0),
                    axis=1, keepdims=True)

    g = _dot_t(x, wg_ref[0])
    u = _dot_t(x, wu_ref[0])
    h = (g * jax.nn.sigmoid(g)) * u
    y = _dot_t(h, wd_ref[0])
    contrib = y * w_col

    @pl.when(e == 0)
    def _init():
        out_ref[...] = contrib

    @pl.when(e > 0)
    def _acc():
        out_ref[...] += contrib


def _shared_kernel(x_ref, sg_ref, su_ref, sd_ref, r_ref, o_ref):
    x = x_ref[...]
    g = _dot_t(x, sg_ref[...])
    u = _dot_t(x, su_ref[...])
    h = (g * jax.nn.sigmoid(g)) * u
    y = _dot_t(h, sd_ref[...])
    o_ref[...] = r_ref[...] + y


def kernel(hidden_states, gate_weight, e_score_correction_bias,
           w_gate, w_up, w_down, shared_gate, shared_up, shared_down):
    x = hidden_states
    bias2d = e_score_correction_bias.reshape(1, N_EXPERTS)

    routed = pl.pallas_call(
        _routed_kernel,
        grid=(N_EXPERTS,),
        in_specs=[
            pl.BlockSpec((T, HIDDEN), lambda e: (0, 0)),
            pl.BlockSpec((N_EXPERTS, HIDDEN), lambda e: (0, 0)),
            pl.BlockSpec((1, N_EXPERTS), lambda e: (0, 0)),
            pl.BlockSpec((1, D_FF, HIDDEN), lambda e: (e, 0, 0)),
            pl.BlockSpec((1, D_FF, HIDDEN), lambda e: (e, 0, 0)),
            pl.BlockSpec((1, HIDDEN, D_FF), lambda e: (e, 0, 0)),
        ],
        out_specs=pl.BlockSpec((T, HIDDEN), lambda e: (0, 0)),
        out_shape=jax.ShapeDtypeStruct((T, HIDDEN), jnp.float32),
        scratch_shapes=[pltpu.VMEM((T, N_EXPERTS), jnp.float32)],
        compiler_params=pltpu.CompilerParams(
            dimension_semantics=("arbitrary",)),
    )(x, gate_weight, bias2d, w_gate, w_up, w_down)

    TB = 512
    out = pl.pallas_call(
        _shared_kernel,
        grid=(T // TB,),
        in_specs=[
            pl.BlockSpec((TB, HIDDEN), lambda i: (i, 0)),
            pl.BlockSpec((D_FF_SHARED, HIDDEN), lambda i: (0, 0)),
            pl.BlockSpec((D_FF_SHARED, HIDDEN), lambda i: (0, 0)),
            pl.BlockSpec((HIDDEN, D_FF_SHARED), lambda i: (0, 0)),
            pl.BlockSpec((TB, HIDDEN), lambda i: (i, 0)),
        ],
        out_specs=pl.BlockSpec((TB, HIDDEN), lambda i: (i, 0)),
        out_shape=jax.ShapeDtypeStruct((T, HIDDEN), jnp.float32),
        compiler_params=pltpu.CompilerParams(
            dimension_semantics=("parallel",)),
    )(x, shared_gate, shared_up, shared_down, routed)

    return out
